# baseline (speedup 1.0000x reference)
import functools

import jax
import jax.numpy as jnp
from jax import lax
from jax.experimental import pallas as pl
from jax.experimental.pallas import tpu as pltpu
import jax.experimental.pallas.tpu_sc as plsc

B = 4
N = 8192
K = 2048
D = 768
ROWS = N // 128
KROWS = K // 128

_NC = 2
_NS = 16
_NW = _NC * _NS
_CHUNK = K // _NW


def _sort_body(sig_ref, out_ref):
    b = pl.program_id(0)
    v = sig_ref[0]
    row = lax.broadcasted_iota(jnp.int32, (ROWS, 128), 0)
    col = lax.broadcasted_iota(jnp.int32, (ROWS, 128), 1)
    flat = row * 128 + col
    idx = flat

    k = 2
    while k <= N:
        j = k // 2
        while j >= 1:
            is_lo = (flat & j) == 0
            desc = (flat & k) == 0
            if j >= 128:
                s = j // 128
                up_v = jnp.roll(v, -s, axis=0)
                dn_v = jnp.roll(v, s, axis=0)
                up_i = jnp.roll(idx, -s, axis=0)
                dn_i = jnp.roll(idx, s, axis=0)
            else:
                up_v = jnp.roll(v, -j, axis=1)
                dn_v = jnp.roll(v, j, axis=1)
                up_i = jnp.roll(idx, -j, axis=1)
                dn_i = jnp.roll(idx, j, axis=1)
            pv = jnp.where(is_lo, up_v, dn_v)
            pi = jnp.where(is_lo, up_i, dn_i)
            sg = (v > pv) | ((v == pv) & (idx < pi))
            ts = sg ^ is_lo ^ desc
            v = jnp.where(ts, v, pv)
            idx = jnp.where(ts, idx, pi)
            j //= 2
        k *= 2

    out_ref[0] = idx[0:KROWS, :] + (b * (N + 1) + 1)


def _topk_indices(significance):
    sig3 = significance.reshape(B, ROWS, 128)
    out = pl.pallas_call(
        _sort_body,
        grid=(B,),
        in_specs=[pl.BlockSpec((1, ROWS, 128), lambda b: (b, 0, 0))],
        out_specs=pl.BlockSpec((1, KROWS, 128), lambda b: (b, 0, 0)),
        out_shape=jax.ShapeDtypeStruct((B, KROWS, 128), jnp.int32),
    )(sig3)
    return out.reshape(B * K)


_SEG = 17 * 128
_CW = 128
_NCHUNK = B * (K // _CW)
_PER_W = _NCHUNK // _NW


def _gather_body(x_hbm, idx_hbm, out_hbm, idx_v, rows_v, idx1_v, row1_v, sem):
    wid = lax.axis_index("s") * _NC + lax.axis_index("c")

    for u in range(_PER_W):
        gc = wid * _PER_W + u
        b = gc // (K // _CW)
        ci = gc % (K // _CW)
        pltpu.sync_copy(idx_hbm.at[pl.ds(b * _SEG + ci * _CW, _CW)], idx_v)
        pltpu.async_copy(x_hbm.at[idx_v], rows_v, sem).wait()
        pltpu.sync_copy(rows_v, out_hbm.at[b, pl.ds(ci * _CW, _CW)])

    @pl.when(wid < B)
    def _tail():
        pltpu.sync_copy(idx_hbm.at[pl.ds(wid * _SEG + K, 1)], idx1_v)
        pltpu.async_copy(x_hbm.at[idx1_v], row1_v, sem).wait()
        pltpu.sync_copy(row1_v, out_hbm.at[wid, pl.ds(K, 1)])


@functools.cache
def _gather_call():
    return functools.partial(
        pl.kernel,
        out_type=jax.ShapeDtypeStruct((B, K + 1, D), jnp.float32),
        mesh=plsc.VectorSubcoreMesh(
            core_axis_name="c", subcore_axis_name="s",
            num_cores=_NC, num_subcores=_NS),
        scratch_types=[
            pltpu.VMEM((_CW,), jnp.int32),
            pltpu.VMEM((_CW, D), jnp.float32),
            pltpu.VMEM((1,), jnp.int32),
            pltpu.VMEM((1, D), jnp.float32),
            pltpu.SemaphoreType.DMA,
        ],
    )(_gather_body)


def kernel(x, significance):
    idx = _topk_indices(significance).reshape(B, K)
    cls_src = (jnp.arange(B, dtype=jnp.int32) * (N + 1))[:, None]
    pad = jnp.zeros((B, _SEG - (K + 1)), dtype=jnp.int32)
    src_idx = jnp.concatenate([cls_src, idx, pad], axis=1).reshape(B * _SEG)
    x_flat = x.reshape(B * (N + 1), D)
    return _gather_call()(x_flat, src_idx)


# speedup vs baseline: 3.3145x; 3.3145x over previous
import functools

import jax
import jax.numpy as jnp
from jax import lax
from jax.experimental import pallas as pl
from jax.experimental.pallas import tpu as pltpu
import jax.experimental.pallas.tpu_sc as plsc

B = 4
N = 8192
K = 2048
D = 768
ROWS = N // 128
KROWS = K // 128

_NC = 2
_NS = 16
_NW = _NC * _NS
_CHUNK = K // _NW


def _sort_body(sig_ref, out_ref):
    b = pl.program_id(0)
    v = sig_ref[0]
    row = lax.broadcasted_iota(jnp.int32, (ROWS, 128), 0)
    col = lax.broadcasted_iota(jnp.int32, (ROWS, 128), 1)
    flat = row * 128 + col
    idx = flat

    k = 2
    while k <= N:
        j = k // 2
        while j >= 1:
            is_lo = (flat & j) == 0
            desc = (flat & k) == 0
            if j >= 128:
                s = j // 128
                up_v = jnp.roll(v, -s, axis=0)
                dn_v = jnp.roll(v, s, axis=0)
                up_i = jnp.roll(idx, -s, axis=0)
                dn_i = jnp.roll(idx, s, axis=0)
            else:
                up_v = jnp.roll(v, -j, axis=1)
                dn_v = jnp.roll(v, j, axis=1)
                up_i = jnp.roll(idx, -j, axis=1)
                dn_i = jnp.roll(idx, j, axis=1)
            pv = jnp.where(is_lo, up_v, dn_v)
            pi = jnp.where(is_lo, up_i, dn_i)
            sg = (v > pv) | ((v == pv) & (idx < pi))
            ts = sg ^ is_lo ^ desc
            v = jnp.where(ts, v, pv)
            idx = jnp.where(ts, idx, pi)
            j //= 2
        k *= 2

    del b
    out_ref[0] = idx[0:KROWS, :] + 1


def _topk_indices(significance):
    sig3 = significance.reshape(B, ROWS, 128)
    out = pl.pallas_call(
        _sort_body,
        grid=(B,),
        in_specs=[pl.BlockSpec((1, ROWS, 128), lambda b: (b, 0, 0))],
        out_specs=pl.BlockSpec((1, KROWS, 128), lambda b: (b, 0, 0)),
        out_shape=jax.ShapeDtypeStruct((B, KROWS, 128), jnp.int32),
    )(sig3)
    return out.reshape(B * K)


_SEG = 17 * 128
_CW = 128
_NCHUNK = B * (K // _CW)
_PER_W = _NCHUNK // _NW


def _gather_body(x_hbm, idx_hbm, out_hbm, idx_v, rows_v, idx1_v, row1_v, sem):
    wid = lax.axis_index("s") * _NC + lax.axis_index("c")

    for u in range(_PER_W):
        gc = wid * _PER_W + u
        b = gc // (K // _CW)
        ci = gc % (K // _CW)
        pltpu.sync_copy(idx_hbm.at[pl.ds(b * _SEG + ci * _CW, _CW)], idx_v)
        pltpu.async_copy(x_hbm.at[b].at[idx_v], rows_v, sem).wait()
        pltpu.sync_copy(rows_v, out_hbm.at[b, pl.ds(ci * _CW, _CW)])

    @pl.when(wid < B)
    def _tail():
        pltpu.sync_copy(idx_hbm.at[pl.ds(wid * _SEG + K, 1)], idx1_v)
        pltpu.async_copy(x_hbm.at[wid].at[idx1_v], row1_v, sem).wait()
        pltpu.sync_copy(row1_v, out_hbm.at[wid, pl.ds(K, 1)])


@functools.cache
def _gather_call():
    return functools.partial(
        pl.kernel,
        out_type=jax.ShapeDtypeStruct((B, K + 1, D), jnp.float32),
        mesh=plsc.VectorSubcoreMesh(
            core_axis_name="c", subcore_axis_name="s",
            num_cores=_NC, num_subcores=_NS),
        scratch_types=[
            pltpu.VMEM((_CW,), jnp.int32),
            pltpu.VMEM((_CW, D), jnp.float32),
            pltpu.VMEM((1,), jnp.int32),
            pltpu.VMEM((1, D), jnp.float32),
            pltpu.SemaphoreType.DMA,
        ],
    )(_gather_body)


def kernel(x, significance):
    idx = _topk_indices(significance).reshape(B, K)
    cls_src = jnp.zeros((B, 1), dtype=jnp.int32)
    pad = jnp.zeros((B, _SEG - (K + 1)), dtype=jnp.int32)
    src_idx = jnp.concatenate([cls_src, idx, pad], axis=1).reshape(B * _SEG)
    return _gather_call()(x, src_idx)


# speedup vs baseline: 3.3210x; 1.0020x over previous
import functools

import jax
import jax.numpy as jnp
from jax import lax
from jax.experimental import pallas as pl
from jax.experimental.pallas import tpu as pltpu
import jax.experimental.pallas.tpu_sc as plsc

B = 4
N = 8192
K = 2048
D = 768
ROWS = N // 128
KROWS = K // 128

_NC = 2
_NS = 16
_NW = _NC * _NS
_CHUNK = K // _NW


def _sort_body(sig_ref, out_ref):
    b = pl.program_id(0)
    v = sig_ref[0]
    row = lax.broadcasted_iota(jnp.int32, (ROWS, 128), 0)
    col = lax.broadcasted_iota(jnp.int32, (ROWS, 128), 1)
    flat = row * 128 + col
    idx = flat

    k = 2
    while k <= N:
        j = k // 2
        while j >= 1:
            is_lo = (flat & j) == 0
            desc = (flat & k) == 0
            if j >= 128:
                s = j // 128
                up_v = jnp.roll(v, -s, axis=0)
                dn_v = jnp.roll(v, s, axis=0)
                up_i = jnp.roll(idx, -s, axis=0)
                dn_i = jnp.roll(idx, s, axis=0)
            else:
                up_v = jnp.roll(v, -j, axis=1)
                dn_v = jnp.roll(v, j, axis=1)
                up_i = jnp.roll(idx, -j, axis=1)
                dn_i = jnp.roll(idx, j, axis=1)
            pv = jnp.where(is_lo, up_v, dn_v)
            pi = jnp.where(is_lo, up_i, dn_i)
            sg = (v > pv) | ((v == pv) & (idx < pi))
            ts = sg ^ is_lo ^ desc
            v = jnp.where(ts, v, pv)
            idx = jnp.where(ts, idx, pi)
            j //= 2
        k *= 2

    del b
    out_ref[0] = idx[0:KROWS, :] + 1


def _topk_indices(significance):
    sig3 = significance.reshape(B, ROWS, 128)
    out = pl.pallas_call(
        _sort_body,
        grid=(B,),
        in_specs=[pl.BlockSpec((1, ROWS, 128), lambda b: (b, 0, 0))],
        out_specs=pl.BlockSpec((1, KROWS, 128), lambda b: (b, 0, 0)),
        out_shape=jax.ShapeDtypeStruct((B, KROWS, 128), jnp.int32),
    )(sig3)
    return out.reshape(B * K)


_SEG = 17 * 128
_CW = 128
_NCHUNK = B * (K // _CW)
_PER_W = _NCHUNK // _NW


def _gather_body(x_hbm, idx_hbm, out_hbm, idx_v, rows_v, idx1_v, row1_v, sem):
    wid = lax.axis_index("s") * _NC + lax.axis_index("c")

    for u in range(_PER_W):
        gc = wid * _PER_W + u
        b = gc // (K // _CW)
        ci = gc % (K // _CW)
        pltpu.sync_copy(idx_hbm.at[pl.ds(b * _SEG + ci * _CW, _CW)], idx_v)
        pltpu.async_copy(x_hbm.at[b].at[idx_v], rows_v, sem).wait()
        pltpu.sync_copy(rows_v, out_hbm.at[b, pl.ds(ci * _CW, _CW)])

    @pl.when(wid < B)
    def _tail():
        pltpu.sync_copy(idx_hbm.at[pl.ds(wid * _SEG + K, 1)], idx1_v)
        pltpu.async_copy(x_hbm.at[wid].at[idx1_v], row1_v, sem).wait()
        pltpu.sync_copy(row1_v, out_hbm.at[wid, pl.ds(K, 1)])


@functools.cache
def _gather_call():
    return functools.partial(
        pl.kernel,
        out_type=jax.ShapeDtypeStruct((B, K + 1, D), jnp.float32),
        mesh=plsc.VectorSubcoreMesh(
            core_axis_name="c", subcore_axis_name="s",
            num_cores=_NC, num_subcores=_NS),
        scratch_types=[
            pltpu.VMEM((_CW,), jnp.int32),
            pltpu.VMEM((_CW, D), jnp.float32),
            pltpu.VMEM((1,), jnp.int32),
            pltpu.VMEM((1, D), jnp.float32),
            pltpu.SemaphoreType.DMA,
        ],
        compiler_params=pltpu.CompilerParams(use_tc_tiling_on_sc=True),
    )(_gather_body)


def kernel(x, significance):
    idx = _topk_indices(significance).reshape(B, K)
    cls_src = jnp.zeros((B, 1), dtype=jnp.int32)
    pad = jnp.zeros((B, _SEG - (K + 1)), dtype=jnp.int32)
    src_idx = jnp.concatenate([cls_src, idx, pad], axis=1).reshape(B * _SEG)
    return _gather_call()(x, src_idx)
